# Initial kernel scaffold; baseline (speedup 1.0000x reference)
import jax
import jax.numpy as jnp
from jax.experimental import pallas as pl


def kernel(s_parent, mask_parent, node_mask, res_idx, Wff, Wq, Wk, Wv, g1, b1, g2, b2):
    raise NotImplementedError("write your pallas kernel here")


# speedup vs baseline: 21.4655x; 21.4655x over previous
import functools
import math

import jax
import jax.numpy as jnp
from jax import lax
from jax.experimental import pallas as pl
from jax.experimental.pallas import tpu as pltpu
from jax.experimental.pallas import tpu_sc as plsc

_TN = 512
_TR = 1024
_CH = 128


def _attn_body(ridx_ref, nm_ref, mpT_ref, sp_ref, wff_ref, wq_ref, wk_ref,
               wv_ref, g1_ref, b1_ref, v_out, wT_out, iT_out, fT_out, k_scr):
    b = pl.program_id(0)
    nb = pl.program_id(1)
    K, C = k_scr.shape
    TN = wT_out.shape[2]

    @pl.when(nb == 0)
    def _():
        sp = sp_ref[0]
        k_scr[...] = lax.dot_general(sp, wk_ref[...], (((1,), (1,)), ((), ())),
                                     preferred_element_type=jnp.float32)
        v_out[0] = lax.dot_general(sp, wv_ref[...], (((1,), (1,)), ((), ())),
                                   preferred_element_type=jnp.float32)

    Lb = jnp.maximum(jnp.sum(nm_ref[pl.ds(b, 1), :]), 1.0)
    denom = jnp.maximum(Lb - 1.0, 1.0)
    m_row = nm_ref[pl.ds(b, 1), pl.ds(nb * TN, TN)]
    pos = jnp.clip(ridx_ref[pl.ds(b, 1), pl.ds(nb * TN, TN)] / denom, 0.0, 1.0)
    projT = (2.0 * math.pi) * (wff_ref[...] * pos)
    q0T = jnp.concatenate([jnp.cos(projT), jnp.sin(projT)], axis=0)
    q0T = q0T * m_row
    mu = jnp.mean(q0T, axis=0, keepdims=True)
    var = jnp.mean((q0T - mu) ** 2, axis=0, keepdims=True)
    q0T = (q0T - mu) * lax.rsqrt(var + 1e-5) * g1_ref[...] + b1_ref[...]

    qT = lax.dot_general(wq_ref[...], q0T, (((1,), (0,)), ((), ())),
                         preferred_element_type=jnp.float32)
    logitsT = lax.dot_general(k_scr[...], qT, (((1,), (0,)), ((), ())),
                              preferred_element_type=jnp.float32)
    logitsT = logitsT * (1.0 / math.sqrt(C))
    logitsT = logitsT + (mpT_ref[0] - 1.0) * 1e9

    iotaK = lax.broadcasted_iota(jnp.int32, (K, TN), 0)
    m1 = jnp.max(logitsT, axis=0, keepdims=True)
    i1 = jnp.min(jnp.where(logitsT == m1, iotaK, K), axis=0, keepdims=True)
    excl = jnp.where(iotaK == i1, -jnp.inf, logitsT)
    m2 = jnp.max(excl, axis=0, keepdims=True)
    i2 = jnp.min(jnp.where(excl == m2, iotaK, K), axis=0, keepdims=True)

    e = jnp.exp(m2 - m1)
    inv_s = m_row / (1.0 + e)
    wT_out[0, pl.ds(0, 1), :] = inv_s
    wT_out[0, pl.ds(1, 1), :] = e * inv_s
    iT_out[0, pl.ds(0, 1), :] = i1
    iT_out[0, pl.ds(1, 1), :] = i2
    off = b * K
    fT_out[0, pl.ds(0, 1), :] = i1 + off
    fT_out[0, pl.ds(1, 1), :] = i2 + off


def _tc_attn(ridxf, nm, mpT, s_parent, wffc, Wq, Wk, Wv, g1c, b1c):
    B, N = ridxf.shape
    _, K, C = s_parent.shape
    TN = _TN
    grid = (B, N // TN)
    full2 = lambda shape: pl.BlockSpec(shape, lambda b, nb: (0, 0))
    out_shape = [
        jax.ShapeDtypeStruct((B, K, C), jnp.float32),
        jax.ShapeDtypeStruct((B, 2, N), jnp.float32),
        jax.ShapeDtypeStruct((B, 2, N), jnp.int32),
        jax.ShapeDtypeStruct((B, 2, N), jnp.int32),
    ]
    in_specs = [
        full2((B, N)),
        full2((B, N)),
        pl.BlockSpec((1, K, 1), lambda b, nb: (b, 0, 0)),
        pl.BlockSpec((1, K, C), lambda b, nb: (b, 0, 0)),
        full2((C // 2, 1)),
        full2((C, C)), full2((C, C)), full2((C, C)),
        full2((C, 1)), full2((C, 1)),
    ]
    out_specs = [
        pl.BlockSpec((1, K, C), lambda b, nb: (b, 0, 0)),
        pl.BlockSpec((1, 2, TN), lambda b, nb: (b, 0, nb)),
        pl.BlockSpec((1, 2, TN), lambda b, nb: (b, 0, nb)),
        pl.BlockSpec((1, 2, TN), lambda b, nb: (b, 0, nb)),
    ]
    return pl.pallas_call(
        _attn_body,
        grid=grid,
        in_specs=in_specs,
        out_specs=out_specs,
        out_shape=out_shape,
        scratch_shapes=[pltpu.VMEM((K, C), jnp.float32)],
        compiler_params=pltpu.CompilerParams(
            dimension_semantics=("arbitrary", "arbitrary")),
    )(ridxf, nm, mpT, s_parent, wffc, Wq, Wk, Wv, g1c, b1c)


def _sc_gather(v2, i0, i1):
    ROWS = i0.shape[0]
    C = v2.shape[1]
    NW = 32
    RPW = ROWS // NW
    CH = _CH
    NCH = RPW // CH

    mesh = plsc.VectorSubcoreMesh(core_axis_name="c", subcore_axis_name="s")

    @functools.partial(
        pl.kernel, mesh=mesh,
        out_type=[jax.ShapeDtypeStruct((ROWS, C), jnp.float32),
                  jax.ShapeDtypeStruct((ROWS, C), jnp.float32)],
        scratch_types=[
            pltpu.VMEM((CH,), jnp.int32), pltpu.VMEM((CH,), jnp.int32),
            pltpu.VMEM((CH, C), jnp.float32), pltpu.VMEM((CH, C), jnp.float32),
            pltpu.SemaphoreType.DMA, pltpu.SemaphoreType.DMA,
        ],
    )
    def body(v2_hbm, i0_hbm, i1_hbm, a_hbm, b_hbm,
             i0_v, i1_v, ra_v, rb_v, semA, semB):
        wid = lax.axis_index("s") * 2 + lax.axis_index("c")

        def chunk(t, _):
            base = wid * RPW + t * CH
            pltpu.sync_copy(i0_hbm.at[pl.ds(base, CH)], i0_v)
            pltpu.sync_copy(i1_hbm.at[pl.ds(base, CH)], i1_v)
            cpA = pltpu.async_copy(v2_hbm.at[i0_v], ra_v, semA)
            cpB = pltpu.async_copy(v2_hbm.at[i1_v], rb_v, semB)
            cpA.wait()
            cpB.wait()
            pltpu.sync_copy(ra_v, a_hbm.at[pl.ds(base, CH)])
            pltpu.sync_copy(rb_v, b_hbm.at[pl.ds(base, CH)])
            return 0

        lax.fori_loop(0, NCH, chunk, 0)

    return body(v2, i0, i1)


def _ln_body(a_ref, b_ref, w0_ref, w1_ref, g2_ref, b2_ref, o_ref):
    s0 = w0_ref[...] * a_ref[...] + w1_ref[...] * b_ref[...]
    mu = jnp.mean(s0, axis=-1, keepdims=True)
    var = jnp.mean((s0 - mu) ** 2, axis=-1, keepdims=True)
    o_ref[...] = (s0 - mu) * lax.rsqrt(var + 1e-5) * g2_ref[...] + b2_ref[...]


def _tc_combine(a, bv, w0c, w1c, g2r, b2r):
    ROWS, C = a.shape
    TR = _TR
    return pl.pallas_call(
        _ln_body,
        grid=(ROWS // TR,),
        in_specs=[
            pl.BlockSpec((TR, C), lambda r: (r, 0)),
            pl.BlockSpec((TR, C), lambda r: (r, 0)),
            pl.BlockSpec((TR, 1), lambda r: (r, 0)),
            pl.BlockSpec((TR, 1), lambda r: (r, 0)),
            pl.BlockSpec((1, C), lambda r: (0, 0)),
            pl.BlockSpec((1, C), lambda r: (0, 0)),
        ],
        out_specs=pl.BlockSpec((TR, C), lambda r: (r, 0)),
        out_shape=jax.ShapeDtypeStruct((ROWS, C), jnp.float32),
        compiler_params=pltpu.CompilerParams(
            dimension_semantics=("arbitrary",)),
    )(a, bv, w0c, w1c, g2r, b2r)


def kernel(s_parent, mask_parent, node_mask, res_idx, Wff, Wq, Wk, Wv,
           g1, b1, g2, b2):
    B, K, C = s_parent.shape
    N = res_idx.shape[1]
    ridxf = res_idx.astype(jnp.float32)
    nm = node_mask.astype(jnp.float32)
    mpT = mask_parent.astype(jnp.float32)[:, :, None]
    wffc = Wff.astype(jnp.float32).T
    g1c = g1[:, None]
    b1c = b1[:, None]

    v, wT, iT, fT = _tc_attn(ridxf, nm, mpT, s_parent, wffc, Wq, Wk, Wv,
                             g1c, b1c)

    v2 = v.reshape(B * K, C)
    i0 = fT[:, 0, :].reshape(-1)
    i1 = fT[:, 1, :].reshape(-1)
    a, bv = _sc_gather(v2, i0, i1)

    w0c = wT[:, 0, :].reshape(-1, 1)
    w1c = wT[:, 1, :].reshape(-1, 1)
    s_flat = _tc_combine(a, bv, w0c, w1c, g2[None, :], b2[None, :])

    s_fine = s_flat.reshape(B, N, C)
    B_local = jnp.transpose(wT, (0, 2, 1))
    parent_idx = jnp.transpose(iT, (0, 2, 1))
    return (s_fine, B_local, parent_idx, jnp.float32(0.0))
